# Initial kernel scaffold; baseline (speedup 1.0000x reference)
import jax
import jax.numpy as jnp
from jax.experimental import pallas as pl


def kernel(x, edge_index, batch_vec, u_idx, v_idx, W1, b1, W2, b2, W3, b3, Wm1, bm1, Wm2, bm2):
    raise NotImplementedError("write your pallas kernel here")


# speedup vs baseline: 3.1885x; 3.1885x over previous
import functools

import jax
import jax.numpy as jnp
from jax import lax
from jax.experimental import pallas as pl
from jax.experimental.pallas import tpu as pltpu
from jax.experimental.pallas import tpu_sc as plsc

N = 10000
E = 320000
B = 512
DIN = 128
H = 256
HH = 128
NPAD = 10240
SENT = N
NC, NS = 2, 16
ECH = 128
NCH = 160
EPAD = NS * NCH * ECH
NHALF = NPAD // 2
ACC_R = NHALF + 8
PARK = NHALF
RPH = NHALF // NS
GPT = 128
RBLK = 512
NBLK = NPAD // RBLK

_f32 = jnp.float32
_i32 = jnp.int32


def _rewrite_dst(dst_v, j, idx_v, base):
    for k in range(ECH // 16):
        dvec = dst_v[j, pl.ds(k * 16, 16)]
        local = dvec - base
        ok = (local >= 0) & (local < NHALF)
        idx_v[pl.ds(k * 16, 16)] = jnp.where(ok, local, PARK)


def _deg_body(dst16, ones_hbm, deg, dst_v, ones_v, idx_v, acc):
    c = lax.axis_index("c")
    s = lax.axis_index("s")
    base = c * NHALF
    row0 = base + s * RPH
    arow0 = s * RPH
    pltpu.sync_copy(dst16.at[s], dst_v)
    pltpu.sync_copy(ones_hbm.at[pl.ds(0, ECH)], ones_v)
    pltpu.sync_copy(ones_hbm.at[pl.ds(0, RPH)], acc.at[pl.ds(arow0, RPH)])
    plsc.subcore_barrier()

    def body(j, carry):
        _rewrite_dst(dst_v, j, idx_v, base)
        pltpu.sync_copy(ones_v, acc.at[idx_v], add=True)
        return carry

    lax.fori_loop(0, NCH, body, 0)
    plsc.subcore_barrier()
    pltpu.sync_copy(acc.at[pl.ds(arow0, RPH)], deg.at[pl.ds(row0, RPH)])


def _scatter_body(ya, yb, src16, dst16, outa, outb,
                  src_v, dst_v, idx_v, gbuf, acc, sem):
    c = lax.axis_index("c")
    s = lax.axis_index("s")
    arow0 = s * RPH
    pltpu.sync_copy(src16.at[s], src_v)
    pltpu.sync_copy(dst16.at[s], dst_v)

    for p in range(2):
        base = p * NHALF
        row0 = base + s * RPH

        @pl.when(c == 0)
        def _i0(row0=row0):
            pltpu.sync_copy(ya.at[pl.ds(row0, RPH)], acc.at[pl.ds(arow0, RPH)])

        @pl.when(c == 1)
        def _i1(row0=row0):
            pltpu.sync_copy(yb.at[pl.ds(row0, RPH)], acc.at[pl.ds(arow0, RPH)])

        plsc.subcore_barrier()

        def body(j, carry, base=base):
            @pl.when(c == 0)
            def _g0():
                pltpu.async_copy(ya.at[src_v.at[j]], gbuf, sem).wait()

            @pl.when(c == 1)
            def _g1():
                pltpu.async_copy(yb.at[src_v.at[j]], gbuf, sem).wait()

            _rewrite_dst(dst_v, j, idx_v, base)
            pltpu.sync_copy(gbuf, acc.at[idx_v], add=True)
            return carry

        lax.fori_loop(0, NCH, body, 0)
        plsc.subcore_barrier()

        @pl.when(c == 0)
        def _o0(row0=row0):
            pltpu.sync_copy(acc.at[pl.ds(arow0, RPH)],
                            outa.at[pl.ds(row0, RPH)])

        @pl.when(c == 1)
        def _o1(row0=row0):
            pltpu.sync_copy(acc.at[pl.ds(arow0, RPH)],
                            outb.at[pl.ds(row0, RPH)])


def _gather_body(h3a, h3b, idx16, outa, outb, idx_v, rows_v, sem):
    c = lax.axis_index("c")
    s = lax.axis_index("s")
    row0 = s * GPT
    pltpu.sync_copy(idx16.at[s], idx_v)

    @pl.when(c == 0)
    def _g0():
        pltpu.async_copy(h3a.at[idx_v], rows_v, sem).wait()
        pltpu.sync_copy(rows_v, outa.at[pl.ds(row0, GPT)])

    @pl.when(c == 1)
    def _g1():
        pltpu.async_copy(h3b.at[idx_v], rows_v, sem).wait()
        pltpu.sync_copy(rows_v, outb.at[pl.ds(row0, GPT)])


@functools.lru_cache(maxsize=1)
def _sc_kernels():
    mesh = plsc.VectorSubcoreMesh(
        core_axis_name="c", subcore_axis_name="s",
        num_cores=NC, num_subcores=NS)
    deg = pl.kernel(
        _deg_body,
        out_type=jax.ShapeDtypeStruct((NPAD, HH), _f32),
        mesh=mesh,
        scratch_types=[
            pltpu.VMEM((NCH, ECH), _i32),
            pltpu.VMEM((ECH, HH), _f32),
            pltpu.VMEM((ECH,), _i32),
            pltpu.VMEM_SHARED((ACC_R, HH), _f32),
        ],
    )
    scat = pl.kernel(
        _scatter_body,
        out_type=(
            jax.ShapeDtypeStruct((NPAD, HH), _f32),
            jax.ShapeDtypeStruct((NPAD, HH), _f32),
        ),
        mesh=mesh,
        scratch_types=[
            pltpu.VMEM((NCH, ECH), _i32),
            pltpu.VMEM((NCH, ECH), _i32),
            pltpu.VMEM((ECH,), _i32),
            pltpu.VMEM((ECH, HH), _f32),
            pltpu.VMEM_SHARED((ACC_R, HH), _f32),
            pltpu.SemaphoreType.DMA,
        ],
    )
    gath = pl.kernel(
        _gather_body,
        out_type=(
            jax.ShapeDtypeStruct((NS * GPT, HH), _f32),
            jax.ShapeDtypeStruct((NS * GPT, HH), _f32),
        ),
        mesh=mesh,
        scratch_types=[
            pltpu.VMEM((GPT,), _i32),
            pltpu.VMEM((GPT, HH), _f32),
            pltpu.SemaphoreType.DMA,
        ],
    )
    return deg, scat, gath


def _tc1_body(deg, x, w1, ya, yb, dis):
    d = lax.rsqrt(deg[:, :1])
    y = jnp.dot(x[...], w1[...], preferred_element_type=_f32) * d
    ya[...] = y[:, :HH]
    yb[...] = y[:, HH:]
    dis[...] = jnp.broadcast_to(d, (RBLK, 16))


_tc1 = pl.pallas_call(
    _tc1_body,
    grid=(NBLK,),
    in_specs=[
        pl.BlockSpec((RBLK, HH), lambda i: (i, 0)),
        pl.BlockSpec((RBLK, DIN), lambda i: (i, 0)),
        pl.BlockSpec((DIN, H), lambda i: (0, 0)),
    ],
    out_specs=[
        pl.BlockSpec((RBLK, HH), lambda i: (i, 0)),
        pl.BlockSpec((RBLK, HH), lambda i: (i, 0)),
        pl.BlockSpec((RBLK, 16), lambda i: (i, 0)),
    ],
    out_shape=[
        jax.ShapeDtypeStruct((NPAD, HH), _f32),
        jax.ShapeDtypeStruct((NPAD, HH), _f32),
        jax.ShapeDtypeStruct((NPAD, 16), _f32),
    ],
)


def _tcmid_body(sa, sb, dis, bias, w, ya, yb):
    d = dis[:, :1]
    ha = jnp.maximum(sa[...] * d + bias[:1, :HH], 0.0)
    hb = jnp.maximum(sb[...] * d + bias[:1, HH:], 0.0)
    h = jnp.concatenate([ha, hb], axis=1)
    y = jnp.dot(h, w[...], preferred_element_type=_f32) * d
    ya[...] = y[:, :HH]
    yb[...] = y[:, HH:]


_tcmid = pl.pallas_call(
    _tcmid_body,
    grid=(NBLK,),
    in_specs=[
        pl.BlockSpec((RBLK, HH), lambda i: (i, 0)),
        pl.BlockSpec((RBLK, HH), lambda i: (i, 0)),
        pl.BlockSpec((RBLK, 16), lambda i: (i, 0)),
        pl.BlockSpec((8, H), lambda i: (0, 0)),
        pl.BlockSpec((H, H), lambda i: (0, 0)),
    ],
    out_specs=[
        pl.BlockSpec((RBLK, HH), lambda i: (i, 0)),
        pl.BlockSpec((RBLK, HH), lambda i: (i, 0)),
    ],
    out_shape=[
        jax.ShapeDtypeStruct((NPAD, HH), _f32),
        jax.ShapeDtypeStruct((NPAD, HH), _f32),
    ],
)


def _tc4_body(sa, sb, dis, bias, batch, h3a_o, h3b_o, gsum_o, cnt_o):
    i = pl.program_id(0)
    d = dis[:, :1]
    ha = sa[...] * d + bias[:1, :HH]
    hb = sb[...] * d + bias[:1, HH:]
    h3a_o[...] = ha
    h3b_o[...] = hb
    bv = batch[:, :1]
    gid = lax.broadcasted_iota(_i32, (RBLK, B), 1)
    oh = (bv == gid).astype(_f32)
    h = jnp.concatenate([ha, hb], axis=1)
    pg = lax.dot_general(oh, h, (((0,), (0,)), ((), ())),
                         preferred_element_type=_f32)
    pc = lax.dot_general(
        oh, jnp.ones((RBLK, 16), _f32), (((0,), (0,)), ((), ())),
        preferred_element_type=_f32,
    )

    @pl.when(i == 0)
    def _z():
        gsum_o[...] = pg
        cnt_o[...] = pc

    @pl.when(i != 0)
    def _a():
        gsum_o[...] += pg
        cnt_o[...] += pc


_tc4 = pl.pallas_call(
    _tc4_body,
    grid=(NBLK,),
    in_specs=[
        pl.BlockSpec((RBLK, HH), lambda i: (i, 0)),
        pl.BlockSpec((RBLK, HH), lambda i: (i, 0)),
        pl.BlockSpec((RBLK, 16), lambda i: (i, 0)),
        pl.BlockSpec((8, H), lambda i: (0, 0)),
        pl.BlockSpec((RBLK, 16), lambda i: (i, 0)),
    ],
    out_specs=[
        pl.BlockSpec((RBLK, HH), lambda i: (i, 0)),
        pl.BlockSpec((RBLK, HH), lambda i: (i, 0)),
        pl.BlockSpec((B, H), lambda i: (0, 0)),
        pl.BlockSpec((B, 16), lambda i: (0, 0)),
    ],
    out_shape=[
        jax.ShapeDtypeStruct((NPAD, HH), _f32),
        jax.ShapeDtypeStruct((NPAD, HH), _f32),
        jax.ShapeDtypeStruct((B, H), _f32),
        jax.ShapeDtypeStruct((B, 16), _f32),
    ],
)


def _tc5_body(gsum, cnt, hua, hub, hva, hvb, wm1, bm1, wm2, bm2, out):
    g = gsum[...] / jnp.maximum(cnt[:, :1], 1.0)
    z = jnp.dot(g, wm1[0:H], preferred_element_type=_f32)
    z += jnp.dot(hua[...], wm1[H:H + HH], preferred_element_type=_f32)
    z += jnp.dot(hub[...], wm1[H + HH:2 * H], preferred_element_type=_f32)
    z += jnp.dot(hva[...], wm1[2 * H:2 * H + HH], preferred_element_type=_f32)
    z += jnp.dot(hvb[...], wm1[2 * H + HH:3 * H], preferred_element_type=_f32)
    z = jnp.maximum(z + bm1[:1, :], 0.0)
    out[...] = jnp.dot(z, wm2[...], preferred_element_type=_f32) + bm2[:1, :]


_tc5 = pl.pallas_call(
    _tc5_body,
    out_shape=jax.ShapeDtypeStruct((B, 128), _f32),
)


@jax.jit
def _impl(x, edge_index, batch_vec, u_idx, v_idx,
          W1, b1, W2, b2, W3, b3, Wm1, bm1, Wm2, bm2):
    src = edge_index[0].astype(_i32)
    dst = edge_index[1].astype(_i32)
    sent = jnp.full((EPAD - E,), SENT, _i32)
    src16 = jnp.concatenate([src, sent]).reshape(NS, NCH, ECH)
    dst16 = jnp.concatenate([dst, sent]).reshape(NS, NCH, ECH)
    xpad = jnp.pad(x.astype(_f32), ((0, NPAD - N), (0, 0)))
    ones_hbm = jnp.ones((NPAD, HH), _f32)

    deg_k, scat_k, gath_k = _sc_kernels()
    deg = deg_k(dst16, ones_hbm)
    y1a, y1b, dis = _tc1(deg, xpad, W1)

    b1b = jnp.broadcast_to(b1[None, :], (8, H))
    b2b = jnp.broadcast_to(b2[None, :], (8, H))
    b3b = jnp.broadcast_to(b3[None, :], (8, H))

    s1a, s1b = scat_k(y1a, y1b, src16, dst16)
    y2a, y2b = _tcmid(s1a, s1b, dis, b1b, W2)
    s2a, s2b = scat_k(y2a, y2b, src16, dst16)
    y3a, y3b = _tcmid(s2a, s2b, dis, b2b, W3)
    s3a, s3b = scat_k(y3a, y3b, src16, dst16)

    batch16 = jnp.broadcast_to(
        jnp.concatenate([batch_vec.astype(_i32),
                         jnp.full((NPAD - N,), B, _i32)])[:, None],
        (NPAD, 16))
    h3a, h3b, gsum, cnt = _tc4(s3a, s3b, dis, b3b, batch16)

    uv = jnp.concatenate(
        [u_idx.astype(_i32), v_idx.astype(_i32),
         jnp.full((NS * GPT - 2 * B,), SENT, _i32)]).reshape(NS, GPT)
    uva, uvb = gath_k(h3a, h3b, uv)

    wm2p = jnp.pad(Wm2, ((0, 0), (0, 127)))
    bm1b = jnp.broadcast_to(bm1[None, :], (8, H))
    bm2b = jnp.broadcast_to(bm2[None, :], (8, 128))
    logit = _tc5(gsum, cnt, uva[:B], uvb[:B], uva[B:2 * B], uvb[B:2 * B],
                 Wm1, bm1b, wm2p, bm2b)
    return logit[:, 0]


def kernel(x, edge_index, batch_vec, u_idx, v_idx,
           W1, b1, W2, b2, W3, b3, Wm1, bm1, Wm2, bm2):
    return _impl(x, edge_index, batch_vec, u_idx, v_idx,
                 W1, b1, W2, b2, W3, b3, Wm1, bm1, Wm2, bm2)


# speedup vs baseline: 3.6819x; 1.1547x over previous
import functools

import jax
import jax.numpy as jnp
from jax import lax
from jax.experimental import pallas as pl
from jax.experimental.pallas import tpu as pltpu
from jax.experimental.pallas import tpu_sc as plsc

N = 10000
E = 320000
B = 512
DIN = 128
H = 256
HH = 128
NPAD = 10240
SENT = N
NC, NS = 2, 16
ECH = 128
NCH = 160
EPAD = NS * NCH * ECH
NHALF = NPAD // 2
ACC_R = NHALF + 8
PARK = NHALF
RPH = NHALF // NS
GPT = 128
RBLK = 512
NBLK = NPAD // RBLK

_f32 = jnp.float32
_i32 = jnp.int32


def _rewrite_dst(dst_v, j, idx_v, base):
    for k in range(ECH // 16):
        dvec = dst_v[j, pl.ds(k * 16, 16)]
        local = dvec - base
        ok = (local >= 0) & (local < NHALF)
        idx_v[pl.ds(k * 16, 16)] = jnp.where(ok, local, PARK)


def _deg_body(dst16, ones_hbm, deg, dst_v, ones_v, idx_v, acc):
    c = lax.axis_index("c")
    s = lax.axis_index("s")
    base = c * NHALF
    row0 = base + s * RPH
    arow0 = s * RPH
    pltpu.sync_copy(dst16.at[s], dst_v)
    pltpu.sync_copy(ones_hbm.at[pl.ds(0, ECH)], ones_v)
    pltpu.sync_copy(ones_hbm.at[pl.ds(0, RPH)], acc.at[pl.ds(arow0, RPH)])
    plsc.subcore_barrier()

    def body(j, carry):
        _rewrite_dst(dst_v, j, idx_v, base)
        pltpu.sync_copy(ones_v, acc.at[idx_v], add=True)
        return carry

    lax.fori_loop(0, NCH, body, 0)
    plsc.subcore_barrier()
    pltpu.sync_copy(acc.at[pl.ds(arow0, RPH)], deg.at[pl.ds(row0, RPH)])


def _scatter_half(y, src_v, dst_v, idx_v, gb0, gb1, sem0, sem1, acc, base):
    gbufs = (gb0, gb1)
    sems = (sem0, sem1)
    pltpu.async_copy(y.at[src_v.at[0]], gb0, sem0)

    def body(i, carry):
        for b in range(2):
            j = 2 * i + b

            @pl.when(j + 1 < NCH)
            def _pf():
                pltpu.async_copy(y.at[src_v.at[j + 1]],
                                 gbufs[1 - b], sems[1 - b])

            pltpu.make_async_copy(y.at[pl.ds(0, ECH)],
                                  gbufs[b], sems[b]).wait()
            _rewrite_dst(dst_v, j, idx_v, base)
            pltpu.sync_copy(gbufs[b], acc.at[idx_v], add=True)
        return carry

    lax.fori_loop(0, NCH // 2, body, 0)


def _scatter_body(ya, yb, src16, dst16, outa, outb,
                  src_v, dst_v, idx_v, gb0, gb1, acc, sem0, sem1):
    c = lax.axis_index("c")
    s = lax.axis_index("s")
    arow0 = s * RPH
    pltpu.sync_copy(src16.at[s], src_v)
    pltpu.sync_copy(dst16.at[s], dst_v)

    for p in range(2):
        base = p * NHALF
        row0 = base + s * RPH

        @pl.when(c == 0)
        def _i0(row0=row0):
            pltpu.sync_copy(ya.at[pl.ds(row0, RPH)], acc.at[pl.ds(arow0, RPH)])

        @pl.when(c == 1)
        def _i1(row0=row0):
            pltpu.sync_copy(yb.at[pl.ds(row0, RPH)], acc.at[pl.ds(arow0, RPH)])

        plsc.subcore_barrier()

        @pl.when(c == 0)
        def _l0(base=base):
            _scatter_half(ya, src_v, dst_v, idx_v, gb0, gb1,
                          sem0, sem1, acc, base)

        @pl.when(c == 1)
        def _l1(base=base):
            _scatter_half(yb, src_v, dst_v, idx_v, gb0, gb1,
                          sem0, sem1, acc, base)

        plsc.subcore_barrier()

        @pl.when(c == 0)
        def _o0(row0=row0):
            pltpu.sync_copy(acc.at[pl.ds(arow0, RPH)],
                            outa.at[pl.ds(row0, RPH)])

        @pl.when(c == 1)
        def _o1(row0=row0):
            pltpu.sync_copy(acc.at[pl.ds(arow0, RPH)],
                            outb.at[pl.ds(row0, RPH)])


def _gather_body(h3a, h3b, idx16, outa, outb, idx_v, rows_v, sem):
    c = lax.axis_index("c")
    s = lax.axis_index("s")
    row0 = s * GPT
    pltpu.sync_copy(idx16.at[s], idx_v)

    @pl.when(c == 0)
    def _g0():
        pltpu.async_copy(h3a.at[idx_v], rows_v, sem).wait()
        pltpu.sync_copy(rows_v, outa.at[pl.ds(row0, GPT)])

    @pl.when(c == 1)
    def _g1():
        pltpu.async_copy(h3b.at[idx_v], rows_v, sem).wait()
        pltpu.sync_copy(rows_v, outb.at[pl.ds(row0, GPT)])


@functools.lru_cache(maxsize=1)
def _sc_kernels():
    mesh = plsc.VectorSubcoreMesh(
        core_axis_name="c", subcore_axis_name="s",
        num_cores=NC, num_subcores=NS)
    deg = pl.kernel(
        _deg_body,
        out_type=jax.ShapeDtypeStruct((NPAD, HH), _f32),
        mesh=mesh,
        scratch_types=[
            pltpu.VMEM((NCH, ECH), _i32),
            pltpu.VMEM((ECH, HH), _f32),
            pltpu.VMEM((ECH,), _i32),
            pltpu.VMEM_SHARED((ACC_R, HH), _f32),
        ],
    )
    scat = pl.kernel(
        _scatter_body,
        out_type=(
            jax.ShapeDtypeStruct((NPAD, HH), _f32),
            jax.ShapeDtypeStruct((NPAD, HH), _f32),
        ),
        mesh=mesh,
        scratch_types=[
            pltpu.VMEM((NCH, ECH), _i32),
            pltpu.VMEM((NCH, ECH), _i32),
            pltpu.VMEM((ECH,), _i32),
            pltpu.VMEM((ECH, HH), _f32),
            pltpu.VMEM((ECH, HH), _f32),
            pltpu.VMEM_SHARED((ACC_R, HH), _f32),
            pltpu.SemaphoreType.DMA,
            pltpu.SemaphoreType.DMA,
        ],
    )
    gath = pl.kernel(
        _gather_body,
        out_type=(
            jax.ShapeDtypeStruct((NS * GPT, HH), _f32),
            jax.ShapeDtypeStruct((NS * GPT, HH), _f32),
        ),
        mesh=mesh,
        scratch_types=[
            pltpu.VMEM((GPT,), _i32),
            pltpu.VMEM((GPT, HH), _f32),
            pltpu.SemaphoreType.DMA,
        ],
    )
    return deg, scat, gath


def _tc1_body(deg, x, w1, ya, yb, dis):
    d = lax.rsqrt(deg[:, :1])
    y = jnp.dot(x[...], w1[...], preferred_element_type=_f32) * d
    ya[...] = y[:, :HH]
    yb[...] = y[:, HH:]
    dis[...] = jnp.broadcast_to(d, (RBLK, 16))


_tc1 = pl.pallas_call(
    _tc1_body,
    grid=(NBLK,),
    in_specs=[
        pl.BlockSpec((RBLK, HH), lambda i: (i, 0)),
        pl.BlockSpec((RBLK, DIN), lambda i: (i, 0)),
        pl.BlockSpec((DIN, H), lambda i: (0, 0)),
    ],
    out_specs=[
        pl.BlockSpec((RBLK, HH), lambda i: (i, 0)),
        pl.BlockSpec((RBLK, HH), lambda i: (i, 0)),
        pl.BlockSpec((RBLK, 16), lambda i: (i, 0)),
    ],
    out_shape=[
        jax.ShapeDtypeStruct((NPAD, HH), _f32),
        jax.ShapeDtypeStruct((NPAD, HH), _f32),
        jax.ShapeDtypeStruct((NPAD, 16), _f32),
    ],
)


def _tcmid_body(sa, sb, dis, bias, w, ya, yb):
    d = dis[:, :1]
    ha = jnp.maximum(sa[...] * d + bias[:1, :HH], 0.0)
    hb = jnp.maximum(sb[...] * d + bias[:1, HH:], 0.0)
    h = jnp.concatenate([ha, hb], axis=1)
    y = jnp.dot(h, w[...], preferred_element_type=_f32) * d
    ya[...] = y[:, :HH]
    yb[...] = y[:, HH:]


_tcmid = pl.pallas_call(
    _tcmid_body,
    grid=(NBLK,),
    in_specs=[
        pl.BlockSpec((RBLK, HH), lambda i: (i, 0)),
        pl.BlockSpec((RBLK, HH), lambda i: (i, 0)),
        pl.BlockSpec((RBLK, 16), lambda i: (i, 0)),
        pl.BlockSpec((8, H), lambda i: (0, 0)),
        pl.BlockSpec((H, H), lambda i: (0, 0)),
    ],
    out_specs=[
        pl.BlockSpec((RBLK, HH), lambda i: (i, 0)),
        pl.BlockSpec((RBLK, HH), lambda i: (i, 0)),
    ],
    out_shape=[
        jax.ShapeDtypeStruct((NPAD, HH), _f32),
        jax.ShapeDtypeStruct((NPAD, HH), _f32),
    ],
)


def _tc4_body(sa, sb, dis, bias, batch, h3a_o, h3b_o, gsum_o, cnt_o):
    i = pl.program_id(0)
    d = dis[:, :1]
    ha = sa[...] * d + bias[:1, :HH]
    hb = sb[...] * d + bias[:1, HH:]
    h3a_o[...] = ha
    h3b_o[...] = hb
    bv = batch[:, :1]
    gid = lax.broadcasted_iota(_i32, (RBLK, B), 1)
    oh = (bv == gid).astype(_f32)
    h = jnp.concatenate([ha, hb], axis=1)
    pg = lax.dot_general(oh, h, (((0,), (0,)), ((), ())),
                         preferred_element_type=_f32)
    pc = lax.dot_general(
        oh, jnp.ones((RBLK, 16), _f32), (((0,), (0,)), ((), ())),
        preferred_element_type=_f32,
    )

    @pl.when(i == 0)
    def _z():
        gsum_o[...] = pg
        cnt_o[...] = pc

    @pl.when(i != 0)
    def _a():
        gsum_o[...] += pg
        cnt_o[...] += pc


_tc4 = pl.pallas_call(
    _tc4_body,
    grid=(NBLK,),
    in_specs=[
        pl.BlockSpec((RBLK, HH), lambda i: (i, 0)),
        pl.BlockSpec((RBLK, HH), lambda i: (i, 0)),
        pl.BlockSpec((RBLK, 16), lambda i: (i, 0)),
        pl.BlockSpec((8, H), lambda i: (0, 0)),
        pl.BlockSpec((RBLK, 16), lambda i: (i, 0)),
    ],
    out_specs=[
        pl.BlockSpec((RBLK, HH), lambda i: (i, 0)),
        pl.BlockSpec((RBLK, HH), lambda i: (i, 0)),
        pl.BlockSpec((B, H), lambda i: (0, 0)),
        pl.BlockSpec((B, 16), lambda i: (0, 0)),
    ],
    out_shape=[
        jax.ShapeDtypeStruct((NPAD, HH), _f32),
        jax.ShapeDtypeStruct((NPAD, HH), _f32),
        jax.ShapeDtypeStruct((B, H), _f32),
        jax.ShapeDtypeStruct((B, 16), _f32),
    ],
)


def _tc5_body(gsum, cnt, hua, hub, hva, hvb, wm1, bm1, wm2, bm2, out):
    g = gsum[...] / jnp.maximum(cnt[:, :1], 1.0)
    z = jnp.dot(g, wm1[0:H], preferred_element_type=_f32)
    z += jnp.dot(hua[...], wm1[H:H + HH], preferred_element_type=_f32)
    z += jnp.dot(hub[...], wm1[H + HH:2 * H], preferred_element_type=_f32)
    z += jnp.dot(hva[...], wm1[2 * H:2 * H + HH], preferred_element_type=_f32)
    z += jnp.dot(hvb[...], wm1[2 * H + HH:3 * H], preferred_element_type=_f32)
    z = jnp.maximum(z + bm1[:1, :], 0.0)
    out[...] = jnp.dot(z, wm2[...], preferred_element_type=_f32) + bm2[:1, :]


_tc5 = pl.pallas_call(
    _tc5_body,
    out_shape=jax.ShapeDtypeStruct((B, 128), _f32),
)


@jax.jit
def _impl(x, edge_index, batch_vec, u_idx, v_idx,
          W1, b1, W2, b2, W3, b3, Wm1, bm1, Wm2, bm2):
    src = edge_index[0].astype(_i32)
    dst = edge_index[1].astype(_i32)
    sent = jnp.full((EPAD - E,), SENT, _i32)
    src16 = jnp.concatenate([src, sent]).reshape(NS, NCH, ECH)
    dst16 = jnp.concatenate([dst, sent]).reshape(NS, NCH, ECH)
    xpad = jnp.pad(x.astype(_f32), ((0, NPAD - N), (0, 0)))
    ones_hbm = jnp.ones((NPAD, HH), _f32)

    deg_k, scat_k, gath_k = _sc_kernels()
    deg = deg_k(dst16, ones_hbm)
    y1a, y1b, dis = _tc1(deg, xpad, W1)

    b1b = jnp.broadcast_to(b1[None, :], (8, H))
    b2b = jnp.broadcast_to(b2[None, :], (8, H))
    b3b = jnp.broadcast_to(b3[None, :], (8, H))

    s1a, s1b = scat_k(y1a, y1b, src16, dst16)
    y2a, y2b = _tcmid(s1a, s1b, dis, b1b, W2)
    s2a, s2b = scat_k(y2a, y2b, src16, dst16)
    y3a, y3b = _tcmid(s2a, s2b, dis, b2b, W3)
    s3a, s3b = scat_k(y3a, y3b, src16, dst16)

    batch16 = jnp.broadcast_to(
        jnp.concatenate([batch_vec.astype(_i32),
                         jnp.full((NPAD - N,), B, _i32)])[:, None],
        (NPAD, 16))
    h3a, h3b, gsum, cnt = _tc4(s3a, s3b, dis, b3b, batch16)

    uv = jnp.concatenate(
        [u_idx.astype(_i32), v_idx.astype(_i32),
         jnp.full((NS * GPT - 2 * B,), SENT, _i32)]).reshape(NS, GPT)
    uva, uvb = gath_k(h3a, h3b, uv)

    wm2p = jnp.pad(Wm2, ((0, 0), (0, 127)))
    bm1b = jnp.broadcast_to(bm1[None, :], (8, H))
    bm2b = jnp.broadcast_to(bm2[None, :], (8, 128))
    logit = _tc5(gsum, cnt, uva[:B], uvb[:B], uva[B:2 * B], uvb[B:2 * B],
                 Wm1, bm1b, wm2p, bm2b)
    return logit[:, 0]


def kernel(x, edge_index, batch_vec, u_idx, v_idx,
           W1, b1, W2, b2, W3, b3, Wm1, bm1, Wm2, bm2):
    return _impl(x, edge_index, batch_vec, u_idx, v_idx,
                 W1, b1, W2, b2, W3, b3, Wm1, bm1, Wm2, bm2)


# speedup vs baseline: 4.0034x; 1.0873x over previous
import functools

import jax
import jax.numpy as jnp
from jax import lax
from jax.experimental import pallas as pl
from jax.experimental.pallas import tpu as pltpu
from jax.experimental.pallas import tpu_sc as plsc

N = 10000
E = 320000
B = 512
DIN = 128
H = 256
HH = 128
NPAD = 10240
SENT = N
NC, NS = 2, 16
ECH = 128
NCH = 160
EPAD = NS * NCH * ECH
NHALF = NPAD // 2
ACC_R = NHALF + 8
PARK = NHALF
RPH = NHALF // NS
GPT = 128
RBLK = 512
NBLK = NPAD // RBLK

_f32 = jnp.float32
_i32 = jnp.int32


def _rewrite_dst(dst_v, j, idx_v, base, park):
    for k in range(ECH // 16):
        dvec = dst_v[j, pl.ds(k * 16, 16)]
        local = dvec - base
        ok = (local >= 0) & (local < NHALF)
        idx_v[pl.ds(k * 16, 16)] = jnp.where(ok, local, park)


def _rewrite_dst1(dst_v, idx_v, base, park):
    for k in range(ECH // 16):
        dvec = dst_v[pl.ds(k * 16, 16)]
        local = dvec - base
        ok = (local >= 0) & (local < NHALF)
        idx_v[pl.ds(k * 16, 16)] = jnp.where(ok, local, park)


def _deg_body(dst16, ones_hbm, deg, st0, st1, ones_v, idx_v, acc,
              dsem0, dsem1):
    c = lax.axis_index("c")
    s = lax.axis_index("s")
    base = c * NHALF
    row0 = base + s * RPH
    arow0 = s * RPH
    park = PARK + lax.rem(s, 8)
    stages = (st0, st1)
    dsems = (dsem0, dsem1)
    pltpu.sync_copy(ones_hbm.at[pl.ds(0, ECH)], ones_v)
    pltpu.sync_copy(ones_hbm.at[pl.ds(0, RPH)], acc.at[pl.ds(arow0, RPH)])
    plsc.subcore_barrier()
    pltpu.async_copy(dst16.at[s].at[0], st0, dsem0)

    def body(i, carry):
        for b in range(2):
            j = 2 * i + b

            @pl.when(j + 1 < NCH)
            def _pf():
                pltpu.async_copy(dst16.at[s].at[j + 1],
                                 stages[1 - b], dsems[1 - b])

            pltpu.make_async_copy(dst16.at[s].at[0],
                                  stages[b], dsems[b]).wait()
            _rewrite_dst1(stages[b], idx_v, base, park)
            pltpu.sync_copy(ones_v, acc.at[idx_v], add=True)
        return carry

    lax.fori_loop(0, NCH // 2, body, 0)
    plsc.subcore_barrier()
    pltpu.sync_copy(acc.at[pl.ds(arow0, RPH)], deg.at[pl.ds(row0, RPH)])


def _scatter_half(y, src_v, dstrow, stages, dsems, idx_v, gbufs, sems,
                  acc, base, park):
    pltpu.async_copy(y.at[src_v.at[0]], gbufs[0], sems[0])
    pltpu.async_copy(dstrow.at[0], stages[0], dsems[0])

    def body(i, carry):
        for b in range(2):
            j = 2 * i + b

            @pl.when(j + 1 < NCH)
            def _pf():
                pltpu.async_copy(y.at[src_v.at[j + 1]],
                                 gbufs[1 - b], sems[1 - b])
                pltpu.async_copy(dstrow.at[j + 1],
                                 stages[1 - b], dsems[1 - b])

            pltpu.make_async_copy(y.at[pl.ds(0, ECH)],
                                  gbufs[b], sems[b]).wait()
            pltpu.make_async_copy(dstrow.at[0],
                                  stages[b], dsems[b]).wait()
            _rewrite_dst1(stages[b], idx_v, base, park)
            pltpu.sync_copy(gbufs[b], acc.at[idx_v], add=True)
        return carry

    lax.fori_loop(0, NCH // 2, body, 0)


def _scatter_body(ya, yb, src16, dst16, outa, outb,
                  src_v, st0, st1, idx_v, gb0, gb1, acc,
                  sem0, sem1, dsem0, dsem1):
    c = lax.axis_index("c")
    s = lax.axis_index("s")
    arow0 = s * RPH
    park = PARK + lax.rem(s, 8)
    gbufs = (gb0, gb1)
    sems = (sem0, sem1)
    stages = (st0, st1)
    dsems = (dsem0, dsem1)
    dstrow = dst16.at[s]
    pltpu.sync_copy(src16.at[s], src_v)

    for p in range(2):
        base = p * NHALF
        row0 = base + s * RPH

        @pl.when(c == 0)
        def _i0(row0=row0):
            pltpu.sync_copy(ya.at[pl.ds(row0, RPH)], acc.at[pl.ds(arow0, RPH)])

        @pl.when(c == 1)
        def _i1(row0=row0):
            pltpu.sync_copy(yb.at[pl.ds(row0, RPH)], acc.at[pl.ds(arow0, RPH)])

        plsc.subcore_barrier()

        @pl.when(c == 0)
        def _l0(base=base):
            _scatter_half(ya, src_v, dstrow, stages, dsems, idx_v,
                          gbufs, sems, acc, base, park)

        @pl.when(c == 1)
        def _l1(base=base):
            _scatter_half(yb, src_v, dstrow, stages, dsems, idx_v,
                          gbufs, sems, acc, base, park)

        plsc.subcore_barrier()

        @pl.when(c == 0)
        def _o0(row0=row0):
            pltpu.sync_copy(acc.at[pl.ds(arow0, RPH)],
                            outa.at[pl.ds(row0, RPH)])

        @pl.when(c == 1)
        def _o1(row0=row0):
            pltpu.sync_copy(acc.at[pl.ds(arow0, RPH)],
                            outb.at[pl.ds(row0, RPH)])


def _gather_body(h3a, h3b, idx8, outa, outb, idx_v, rows_v, sem):
    c = lax.axis_index("c")
    s = lax.axis_index("s")

    @pl.when(s < 8)
    def _active():
        pltpu.sync_copy(idx8.at[s], idx_v)

        @pl.when(c == 0)
        def _g0():
            for t in range(2):
                row0 = s * GPT + t * (GPT // 2)
                pltpu.async_copy(
                    h3a.at[idx_v.at[pl.ds(t * (GPT // 2), GPT // 2)]],
                    rows_v, sem).wait()
                pltpu.sync_copy(rows_v, outa.at[pl.ds(row0, GPT // 2)])

        @pl.when(c == 1)
        def _g1():
            for t in range(2):
                row0 = s * GPT + t * (GPT // 2)
                pltpu.async_copy(
                    h3b.at[idx_v.at[pl.ds(t * (GPT // 2), GPT // 2)]],
                    rows_v, sem).wait()
                pltpu.sync_copy(rows_v, outb.at[pl.ds(row0, GPT // 2)])


@functools.lru_cache(maxsize=1)
def _sc_kernels():
    mesh = plsc.VectorSubcoreMesh(
        core_axis_name="c", subcore_axis_name="s",
        num_cores=NC, num_subcores=NS)
    deg = pl.kernel(
        _deg_body,
        out_type=jax.ShapeDtypeStruct((NPAD, HH), _f32),
        mesh=mesh,
        scratch_types=[
            pltpu.VMEM((ECH,), _i32),
            pltpu.VMEM((ECH,), _i32),
            pltpu.VMEM((ECH, HH), _f32),
            pltpu.VMEM((ECH,), _i32),
            pltpu.VMEM_SHARED((ACC_R, HH), _f32),
            pltpu.SemaphoreType.DMA,
            pltpu.SemaphoreType.DMA,
        ],
    )
    scat = pl.kernel(
        _scatter_body,
        out_type=(
            jax.ShapeDtypeStruct((NPAD, HH), _f32),
            jax.ShapeDtypeStruct((NPAD, HH), _f32),
        ),
        mesh=mesh,
        scratch_types=[
            pltpu.VMEM((NCH, ECH), _i32),
            pltpu.VMEM((ECH,), _i32),
            pltpu.VMEM((ECH,), _i32),
            pltpu.VMEM((ECH,), _i32),
            pltpu.VMEM((ECH, HH), _f32),
            pltpu.VMEM((ECH, HH), _f32),
            pltpu.VMEM_SHARED((ACC_R, HH), _f32),
            pltpu.SemaphoreType.DMA,
            pltpu.SemaphoreType.DMA,
            pltpu.SemaphoreType.DMA,
            pltpu.SemaphoreType.DMA,
        ],
    )
    gath = pl.kernel(
        _gather_body,
        out_type=(
            jax.ShapeDtypeStruct((8 * GPT, HH), _f32),
            jax.ShapeDtypeStruct((8 * GPT, HH), _f32),
        ),
        mesh=mesh,
        scratch_types=[
            pltpu.VMEM((GPT,), _i32),
            pltpu.VMEM((GPT // 2, HH), _f32),
            pltpu.SemaphoreType.DMA,
        ],
    )
    return deg, scat, gath


def _tc1_body(deg, x, w1, ya, yb, dis):
    d = lax.rsqrt(deg[:, :1])
    y = jnp.dot(x[...], w1[...], preferred_element_type=_f32) * d
    ya[...] = y[:, :HH]
    yb[...] = y[:, HH:]
    dis[...] = jnp.broadcast_to(d, (RBLK, 16))


_tc1 = pl.pallas_call(
    _tc1_body,
    grid=(NBLK,),
    in_specs=[
        pl.BlockSpec((RBLK, HH), lambda i: (i, 0)),
        pl.BlockSpec((RBLK, DIN), lambda i: (i, 0)),
        pl.BlockSpec((DIN, H), lambda i: (0, 0)),
    ],
    out_specs=[
        pl.BlockSpec((RBLK, HH), lambda i: (i, 0)),
        pl.BlockSpec((RBLK, HH), lambda i: (i, 0)),
        pl.BlockSpec((RBLK, 16), lambda i: (i, 0)),
    ],
    out_shape=[
        jax.ShapeDtypeStruct((NPAD, HH), _f32),
        jax.ShapeDtypeStruct((NPAD, HH), _f32),
        jax.ShapeDtypeStruct((NPAD, 16), _f32),
    ],
)


def _tcmid_body(sa, sb, dis, bias, w, ya, yb):
    d = dis[:, :1]
    ha = jnp.maximum(sa[...] * d + bias[:1, :HH], 0.0)
    hb = jnp.maximum(sb[...] * d + bias[:1, HH:], 0.0)
    h = jnp.concatenate([ha, hb], axis=1)
    y = jnp.dot(h, w[...], preferred_element_type=_f32) * d
    ya[...] = y[:, :HH]
    yb[...] = y[:, HH:]


_tcmid = pl.pallas_call(
    _tcmid_body,
    grid=(NBLK,),
    in_specs=[
        pl.BlockSpec((RBLK, HH), lambda i: (i, 0)),
        pl.BlockSpec((RBLK, HH), lambda i: (i, 0)),
        pl.BlockSpec((RBLK, 16), lambda i: (i, 0)),
        pl.BlockSpec((8, H), lambda i: (0, 0)),
        pl.BlockSpec((H, H), lambda i: (0, 0)),
    ],
    out_specs=[
        pl.BlockSpec((RBLK, HH), lambda i: (i, 0)),
        pl.BlockSpec((RBLK, HH), lambda i: (i, 0)),
    ],
    out_shape=[
        jax.ShapeDtypeStruct((NPAD, HH), _f32),
        jax.ShapeDtypeStruct((NPAD, HH), _f32),
    ],
)


def _tc4_body(sa, sb, dis, bias, batch, h3a_o, h3b_o, gsum_o, cnt_o):
    i = pl.program_id(0)
    d = dis[:, :1]
    ha = sa[...] * d + bias[:1, :HH]
    hb = sb[...] * d + bias[:1, HH:]
    h3a_o[...] = ha
    h3b_o[...] = hb
    bv = batch[:, :1]
    gid = lax.broadcasted_iota(_i32, (RBLK, B), 1)
    oh = (bv == gid).astype(_f32)
    h = jnp.concatenate([ha, hb], axis=1)
    pg = lax.dot_general(oh, h, (((0,), (0,)), ((), ())),
                         preferred_element_type=_f32)
    pc = lax.dot_general(
        oh, jnp.ones((RBLK, 16), _f32), (((0,), (0,)), ((), ())),
        preferred_element_type=_f32,
    )

    @pl.when(i == 0)
    def _z():
        gsum_o[...] = pg
        cnt_o[...] = pc

    @pl.when(i != 0)
    def _a():
        gsum_o[...] += pg
        cnt_o[...] += pc


_tc4 = pl.pallas_call(
    _tc4_body,
    grid=(NBLK,),
    in_specs=[
        pl.BlockSpec((RBLK, HH), lambda i: (i, 0)),
        pl.BlockSpec((RBLK, HH), lambda i: (i, 0)),
        pl.BlockSpec((RBLK, 16), lambda i: (i, 0)),
        pl.BlockSpec((8, H), lambda i: (0, 0)),
        pl.BlockSpec((RBLK, 16), lambda i: (i, 0)),
    ],
    out_specs=[
        pl.BlockSpec((RBLK, HH), lambda i: (i, 0)),
        pl.BlockSpec((RBLK, HH), lambda i: (i, 0)),
        pl.BlockSpec((B, H), lambda i: (0, 0)),
        pl.BlockSpec((B, 16), lambda i: (0, 0)),
    ],
    out_shape=[
        jax.ShapeDtypeStruct((NPAD, HH), _f32),
        jax.ShapeDtypeStruct((NPAD, HH), _f32),
        jax.ShapeDtypeStruct((B, H), _f32),
        jax.ShapeDtypeStruct((B, 16), _f32),
    ],
)


def _tc5_body(gsum, cnt, hua, hub, hva, hvb, wm1, bm1, wm2, bm2, out):
    g = gsum[...] / jnp.maximum(cnt[:, :1], 1.0)
    z = jnp.dot(g, wm1[0:H], preferred_element_type=_f32)
    z += jnp.dot(hua[...], wm1[H:H + HH], preferred_element_type=_f32)
    z += jnp.dot(hub[...], wm1[H + HH:2 * H], preferred_element_type=_f32)
    z += jnp.dot(hva[...], wm1[2 * H:2 * H + HH], preferred_element_type=_f32)
    z += jnp.dot(hvb[...], wm1[2 * H + HH:3 * H], preferred_element_type=_f32)
    z = jnp.maximum(z + bm1[:1, :], 0.0)
    out[...] = jnp.dot(z, wm2[...], preferred_element_type=_f32) + bm2[:1, :]


_tc5 = pl.pallas_call(
    _tc5_body,
    out_shape=jax.ShapeDtypeStruct((B, 128), _f32),
)


@jax.jit
def _impl(x, edge_index, batch_vec, u_idx, v_idx,
          W1, b1, W2, b2, W3, b3, Wm1, bm1, Wm2, bm2):
    src = edge_index[0].astype(_i32)
    dst = edge_index[1].astype(_i32)
    sent = jnp.full((EPAD - E,), SENT, _i32)
    src16 = jnp.concatenate([src, sent]).reshape(NS, NCH, ECH)
    dst16 = jnp.concatenate([dst, sent]).reshape(NS, NCH, ECH)
    xpad = jnp.pad(x.astype(_f32), ((0, NPAD - N), (0, 0)))
    ones_hbm = jnp.ones((NPAD, HH), _f32)

    deg_k, scat_k, gath_k = _sc_kernels()
    deg = deg_k(dst16, ones_hbm)
    y1a, y1b, dis = _tc1(deg, xpad, W1)

    b1b = jnp.broadcast_to(b1[None, :], (8, H))
    b2b = jnp.broadcast_to(b2[None, :], (8, H))
    b3b = jnp.broadcast_to(b3[None, :], (8, H))

    s1a, s1b = scat_k(y1a, y1b, src16, dst16)
    y2a, y2b = _tcmid(s1a, s1b, dis, b1b, W2)
    s2a, s2b = scat_k(y2a, y2b, src16, dst16)
    y3a, y3b = _tcmid(s2a, s2b, dis, b2b, W3)
    s3a, s3b = scat_k(y3a, y3b, src16, dst16)

    batch16 = jnp.broadcast_to(
        jnp.concatenate([batch_vec.astype(_i32),
                         jnp.full((NPAD - N,), B, _i32)])[:, None],
        (NPAD, 16))
    h3a, h3b, gsum, cnt = _tc4(s3a, s3b, dis, b3b, batch16)

    uv = jnp.concatenate(
        [u_idx.astype(_i32), v_idx.astype(_i32)]).reshape(8, GPT)
    uva, uvb = gath_k(h3a, h3b, uv)

    wm2p = jnp.pad(Wm2, ((0, 0), (0, 127)))
    bm1b = jnp.broadcast_to(bm1[None, :], (8, H))
    bm2b = jnp.broadcast_to(bm2[None, :], (8, 128))
    logit = _tc5(gsum, cnt, uva[:B], uvb[:B], uva[B:2 * B], uvb[B:2 * B],
                 Wm1, bm1b, wm2p, bm2b)
    return logit[:, 0]


def kernel(x, edge_index, batch_vec, u_idx, v_idx,
           W1, b1, W2, b2, W3, b3, Wm1, bm1, Wm2, bm2):
    return _impl(x, edge_index, batch_vec, u_idx, v_idx,
                 W1, b1, W2, b2, W3, b3, Wm1, bm1, Wm2, bm2)


# speedup vs baseline: 4.0609x; 1.0144x over previous
import functools

import jax
import jax.numpy as jnp
from jax import lax
from jax.experimental import pallas as pl
from jax.experimental.pallas import tpu as pltpu
from jax.experimental.pallas import tpu_sc as plsc

N = 10000
E = 320000
B = 512
DIN = 128
H = 256
HH = 128
NPAD = 10240
SENT = N
NC, NS = 2, 16
ECH = 128
NCH = 160
EPAD = NS * NCH * ECH
NHALF = NPAD // 2
ACC_R = NHALF + 8
PARK = NHALF
RPH = NHALF // NS
GPT = 128
RBLK = 512
NBLK = NPAD // RBLK

_f32 = jnp.float32
_i32 = jnp.int32


def _rewrite_dst(dst_v, j, idx_v, base, park):
    for k in range(ECH // 16):
        dvec = dst_v[j, pl.ds(k * 16, 16)]
        local = dvec - base
        ok = (local >= 0) & (local < NHALF)
        idx_v[pl.ds(k * 16, 16)] = jnp.where(ok, local, park)


def _rewrite_dst1(dst_v, idx_v, base, park):
    for k in range(ECH // 16):
        dvec = dst_v[pl.ds(k * 16, 16)]
        local = dvec - base
        ok = (local >= 0) & (local < NHALF)
        idx_v[pl.ds(k * 16, 16)] = jnp.where(ok, local, park)


def _deg_body(dst16, ones_hbm, deg, st0, st1, ones_v, idx_v, acc,
              dsem0, dsem1):
    c = lax.axis_index("c")
    s = lax.axis_index("s")
    base = c * NHALF
    row0 = base + s * RPH
    arow0 = s * RPH
    park = PARK + lax.rem(s, 8)
    stages = (st0, st1)
    dsems = (dsem0, dsem1)
    pltpu.sync_copy(ones_hbm.at[pl.ds(0, ECH)], ones_v)
    pltpu.sync_copy(ones_hbm.at[pl.ds(0, RPH)], acc.at[pl.ds(arow0, RPH)])
    plsc.subcore_barrier()
    pltpu.async_copy(dst16.at[s].at[0], st0, dsem0)

    def body(i, carry):
        for b in range(2):
            j = 2 * i + b

            @pl.when(j + 1 < NCH)
            def _pf():
                pltpu.async_copy(dst16.at[s].at[j + 1],
                                 stages[1 - b], dsems[1 - b])

            pltpu.make_async_copy(dst16.at[s].at[0],
                                  stages[b], dsems[b]).wait()
            _rewrite_dst1(stages[b], idx_v, base, park)
            pltpu.sync_copy(ones_v, acc.at[idx_v], add=True)
        return carry

    lax.fori_loop(0, NCH // 2, body, 0)
    plsc.subcore_barrier()
    pltpu.sync_copy(acc.at[pl.ds(arow0, RPH)], deg.at[pl.ds(row0, RPH)])


NBUF = 3


def _scatter_half(y, src_v, dstrow, stages, dsems, idx_v, gbufs, sems,
                  acc, base, park):
    for j in range(NBUF - 1):
        pltpu.async_copy(y.at[src_v.at[j]], gbufs[j], sems[j])
        pltpu.async_copy(dstrow.at[j], stages[j], dsems[j])

    def _chunk(j, b, prefetch):
        if prefetch:
            @pl.when(j + NBUF - 1 < NCH)
            def _pf():
                pltpu.async_copy(y.at[src_v.at[j + NBUF - 1]],
                                 gbufs[(b + NBUF - 1) % NBUF],
                                 sems[(b + NBUF - 1) % NBUF])
                pltpu.async_copy(dstrow.at[j + NBUF - 1],
                                 stages[(b + NBUF - 1) % NBUF],
                                 dsems[(b + NBUF - 1) % NBUF])

        pltpu.make_async_copy(y.at[pl.ds(0, ECH)],
                              gbufs[b], sems[b]).wait()
        pltpu.make_async_copy(dstrow.at[0],
                              stages[b], dsems[b]).wait()
        _rewrite_dst1(stages[b], idx_v, base, park)
        pltpu.sync_copy(gbufs[b], acc.at[idx_v], add=True)

    def body(i, carry):
        for b in range(NBUF):
            _chunk(NBUF * i + b, b, True)
        return carry

    nfull = NCH // NBUF
    lax.fori_loop(0, nfull, body, 0)
    for j in range(nfull * NBUF, NCH):
        _chunk(j, j % NBUF, False)


def _scatter_body(ya, yb, src16, dst16, outa, outb,
                  src_v, st0, st1, st2, idx_v, gb0, gb1, gb2, acc,
                  sem0, sem1, sem2, dsem0, dsem1, dsem2):
    c = lax.axis_index("c")
    s = lax.axis_index("s")
    arow0 = s * RPH
    park = PARK + lax.rem(s, 8)
    gbufs = (gb0, gb1, gb2)
    sems = (sem0, sem1, sem2)
    stages = (st0, st1, st2)
    dsems = (dsem0, dsem1, dsem2)
    dstrow = dst16.at[s]
    pltpu.sync_copy(src16.at[s], src_v)

    for p in range(2):
        base = p * NHALF
        row0 = base + s * RPH

        @pl.when(c == 0)
        def _i0(row0=row0):
            pltpu.sync_copy(ya.at[pl.ds(row0, RPH)], acc.at[pl.ds(arow0, RPH)])

        @pl.when(c == 1)
        def _i1(row0=row0):
            pltpu.sync_copy(yb.at[pl.ds(row0, RPH)], acc.at[pl.ds(arow0, RPH)])

        plsc.subcore_barrier()

        @pl.when(c == 0)
        def _l0(base=base):
            _scatter_half(ya, src_v, dstrow, stages, dsems, idx_v,
                          gbufs, sems, acc, base, park)

        @pl.when(c == 1)
        def _l1(base=base):
            _scatter_half(yb, src_v, dstrow, stages, dsems, idx_v,
                          gbufs, sems, acc, base, park)

        plsc.subcore_barrier()

        @pl.when(c == 0)
        def _o0(row0=row0):
            pltpu.sync_copy(acc.at[pl.ds(arow0, RPH)],
                            outa.at[pl.ds(row0, RPH)])

        @pl.when(c == 1)
        def _o1(row0=row0):
            pltpu.sync_copy(acc.at[pl.ds(arow0, RPH)],
                            outb.at[pl.ds(row0, RPH)])


def _gather_body(h3a, h3b, idx8, outa, outb, idx_v, rows_v, sem):
    c = lax.axis_index("c")
    s = lax.axis_index("s")

    @pl.when(s < 8)
    def _active():
        pltpu.sync_copy(idx8.at[s], idx_v)

        @pl.when(c == 0)
        def _g0():
            for t in range(2):
                row0 = s * GPT + t * (GPT // 2)
                pltpu.async_copy(
                    h3a.at[idx_v.at[pl.ds(t * (GPT // 2), GPT // 2)]],
                    rows_v, sem).wait()
                pltpu.sync_copy(rows_v, outa.at[pl.ds(row0, GPT // 2)])

        @pl.when(c == 1)
        def _g1():
            for t in range(2):
                row0 = s * GPT + t * (GPT // 2)
                pltpu.async_copy(
                    h3b.at[idx_v.at[pl.ds(t * (GPT // 2), GPT // 2)]],
                    rows_v, sem).wait()
                pltpu.sync_copy(rows_v, outb.at[pl.ds(row0, GPT // 2)])


@functools.lru_cache(maxsize=1)
def _sc_kernels():
    mesh = plsc.VectorSubcoreMesh(
        core_axis_name="c", subcore_axis_name="s",
        num_cores=NC, num_subcores=NS)
    deg = pl.kernel(
        _deg_body,
        out_type=jax.ShapeDtypeStruct((NPAD, HH), _f32),
        mesh=mesh,
        scratch_types=[
            pltpu.VMEM((ECH,), _i32),
            pltpu.VMEM((ECH,), _i32),
            pltpu.VMEM((ECH, HH), _f32),
            pltpu.VMEM((ECH,), _i32),
            pltpu.VMEM_SHARED((ACC_R, HH), _f32),
            pltpu.SemaphoreType.DMA,
            pltpu.SemaphoreType.DMA,
        ],
    )
    scat = pl.kernel(
        _scatter_body,
        out_type=(
            jax.ShapeDtypeStruct((NPAD, HH), _f32),
            jax.ShapeDtypeStruct((NPAD, HH), _f32),
        ),
        mesh=mesh,
        scratch_types=[
            pltpu.VMEM((NCH, ECH), _i32),
            pltpu.VMEM((ECH,), _i32),
            pltpu.VMEM((ECH,), _i32),
            pltpu.VMEM((ECH,), _i32),
            pltpu.VMEM((ECH,), _i32),
            pltpu.VMEM((ECH, HH), _f32),
            pltpu.VMEM((ECH, HH), _f32),
            pltpu.VMEM((ECH, HH), _f32),
            pltpu.VMEM_SHARED((ACC_R, HH), _f32),
            pltpu.SemaphoreType.DMA,
            pltpu.SemaphoreType.DMA,
            pltpu.SemaphoreType.DMA,
            pltpu.SemaphoreType.DMA,
            pltpu.SemaphoreType.DMA,
            pltpu.SemaphoreType.DMA,
        ],
    )
    gath = pl.kernel(
        _gather_body,
        out_type=(
            jax.ShapeDtypeStruct((8 * GPT, HH), _f32),
            jax.ShapeDtypeStruct((8 * GPT, HH), _f32),
        ),
        mesh=mesh,
        scratch_types=[
            pltpu.VMEM((GPT,), _i32),
            pltpu.VMEM((GPT // 2, HH), _f32),
            pltpu.SemaphoreType.DMA,
        ],
    )
    return deg, scat, gath


def _tc1_body(deg, x, w1, ya, yb, dis):
    d = lax.rsqrt(deg[:, :1])
    y = jnp.dot(x[...], w1[...], preferred_element_type=_f32) * d
    ya[...] = y[:, :HH]
    yb[...] = y[:, HH:]
    dis[...] = jnp.broadcast_to(d, (RBLK, 16))


_tc1 = pl.pallas_call(
    _tc1_body,
    grid=(NBLK,),
    in_specs=[
        pl.BlockSpec((RBLK, HH), lambda i: (i, 0)),
        pl.BlockSpec((RBLK, DIN), lambda i: (i, 0)),
        pl.BlockSpec((DIN, H), lambda i: (0, 0)),
    ],
    out_specs=[
        pl.BlockSpec((RBLK, HH), lambda i: (i, 0)),
        pl.BlockSpec((RBLK, HH), lambda i: (i, 0)),
        pl.BlockSpec((RBLK, 16), lambda i: (i, 0)),
    ],
    out_shape=[
        jax.ShapeDtypeStruct((NPAD, HH), _f32),
        jax.ShapeDtypeStruct((NPAD, HH), _f32),
        jax.ShapeDtypeStruct((NPAD, 16), _f32),
    ],
)


def _tcmid_body(sa, sb, dis, bias, w, ya, yb):
    d = dis[:, :1]
    ha = jnp.maximum(sa[...] * d + bias[:1, :HH], 0.0)
    hb = jnp.maximum(sb[...] * d + bias[:1, HH:], 0.0)
    h = jnp.concatenate([ha, hb], axis=1)
    y = jnp.dot(h, w[...], preferred_element_type=_f32) * d
    ya[...] = y[:, :HH]
    yb[...] = y[:, HH:]


_tcmid = pl.pallas_call(
    _tcmid_body,
    grid=(NBLK,),
    in_specs=[
        pl.BlockSpec((RBLK, HH), lambda i: (i, 0)),
        pl.BlockSpec((RBLK, HH), lambda i: (i, 0)),
        pl.BlockSpec((RBLK, 16), lambda i: (i, 0)),
        pl.BlockSpec((8, H), lambda i: (0, 0)),
        pl.BlockSpec((H, H), lambda i: (0, 0)),
    ],
    out_specs=[
        pl.BlockSpec((RBLK, HH), lambda i: (i, 0)),
        pl.BlockSpec((RBLK, HH), lambda i: (i, 0)),
    ],
    out_shape=[
        jax.ShapeDtypeStruct((NPAD, HH), _f32),
        jax.ShapeDtypeStruct((NPAD, HH), _f32),
    ],
)


def _tc4_body(sa, sb, dis, bias, batch, h3a_o, h3b_o, gsum_o, cnt_o):
    i = pl.program_id(0)
    d = dis[:, :1]
    ha = sa[...] * d + bias[:1, :HH]
    hb = sb[...] * d + bias[:1, HH:]
    h3a_o[...] = ha
    h3b_o[...] = hb
    bv = batch[:, :1]
    gid = lax.broadcasted_iota(_i32, (RBLK, B), 1)
    oh = (bv == gid).astype(_f32)
    h = jnp.concatenate([ha, hb], axis=1)
    pg = lax.dot_general(oh, h, (((0,), (0,)), ((), ())),
                         preferred_element_type=_f32)
    pc = lax.dot_general(
        oh, jnp.ones((RBLK, 16), _f32), (((0,), (0,)), ((), ())),
        preferred_element_type=_f32,
    )

    @pl.when(i == 0)
    def _z():
        gsum_o[...] = pg
        cnt_o[...] = pc

    @pl.when(i != 0)
    def _a():
        gsum_o[...] += pg
        cnt_o[...] += pc


_tc4 = pl.pallas_call(
    _tc4_body,
    grid=(NBLK,),
    in_specs=[
        pl.BlockSpec((RBLK, HH), lambda i: (i, 0)),
        pl.BlockSpec((RBLK, HH), lambda i: (i, 0)),
        pl.BlockSpec((RBLK, 16), lambda i: (i, 0)),
        pl.BlockSpec((8, H), lambda i: (0, 0)),
        pl.BlockSpec((RBLK, 16), lambda i: (i, 0)),
    ],
    out_specs=[
        pl.BlockSpec((RBLK, HH), lambda i: (i, 0)),
        pl.BlockSpec((RBLK, HH), lambda i: (i, 0)),
        pl.BlockSpec((B, H), lambda i: (0, 0)),
        pl.BlockSpec((B, 16), lambda i: (0, 0)),
    ],
    out_shape=[
        jax.ShapeDtypeStruct((NPAD, HH), _f32),
        jax.ShapeDtypeStruct((NPAD, HH), _f32),
        jax.ShapeDtypeStruct((B, H), _f32),
        jax.ShapeDtypeStruct((B, 16), _f32),
    ],
)


def _tc5_body(gsum, cnt, hua, hub, hva, hvb, wm1, bm1, wm2, bm2, out):
    g = gsum[...] / jnp.maximum(cnt[:, :1], 1.0)
    z = jnp.dot(g, wm1[0:H], preferred_element_type=_f32)
    z += jnp.dot(hua[...], wm1[H:H + HH], preferred_element_type=_f32)
    z += jnp.dot(hub[...], wm1[H + HH:2 * H], preferred_element_type=_f32)
    z += jnp.dot(hva[...], wm1[2 * H:2 * H + HH], preferred_element_type=_f32)
    z += jnp.dot(hvb[...], wm1[2 * H + HH:3 * H], preferred_element_type=_f32)
    z = jnp.maximum(z + bm1[:1, :], 0.0)
    out[...] = jnp.dot(z, wm2[...], preferred_element_type=_f32) + bm2[:1, :]


_tc5 = pl.pallas_call(
    _tc5_body,
    out_shape=jax.ShapeDtypeStruct((B, 128), _f32),
)


@jax.jit
def _impl(x, edge_index, batch_vec, u_idx, v_idx,
          W1, b1, W2, b2, W3, b3, Wm1, bm1, Wm2, bm2):
    src = edge_index[0].astype(_i32)
    dst = edge_index[1].astype(_i32)
    sent = jnp.full((EPAD - E,), SENT, _i32)
    src16 = jnp.concatenate([src, sent]).reshape(NS, NCH, ECH)
    dst16 = jnp.concatenate([dst, sent]).reshape(NS, NCH, ECH)
    xpad = jnp.pad(x.astype(_f32), ((0, NPAD - N), (0, 0)))
    ones_hbm = jnp.ones((NPAD, HH), _f32)

    deg_k, scat_k, gath_k = _sc_kernels()
    deg = deg_k(dst16, ones_hbm)
    y1a, y1b, dis = _tc1(deg, xpad, W1)

    b1b = jnp.broadcast_to(b1[None, :], (8, H))
    b2b = jnp.broadcast_to(b2[None, :], (8, H))
    b3b = jnp.broadcast_to(b3[None, :], (8, H))

    s1a, s1b = scat_k(y1a, y1b, src16, dst16)
    y2a, y2b = _tcmid(s1a, s1b, dis, b1b, W2)
    s2a, s2b = scat_k(y2a, y2b, src16, dst16)
    y3a, y3b = _tcmid(s2a, s2b, dis, b2b, W3)
    s3a, s3b = scat_k(y3a, y3b, src16, dst16)

    batch16 = jnp.broadcast_to(
        jnp.concatenate([batch_vec.astype(_i32),
                         jnp.full((NPAD - N,), B, _i32)])[:, None],
        (NPAD, 16))
    h3a, h3b, gsum, cnt = _tc4(s3a, s3b, dis, b3b, batch16)

    uv = jnp.concatenate(
        [u_idx.astype(_i32), v_idx.astype(_i32)]).reshape(8, GPT)
    uva, uvb = gath_k(h3a, h3b, uv)

    wm2p = jnp.pad(Wm2, ((0, 0), (0, 127)))
    bm1b = jnp.broadcast_to(bm1[None, :], (8, H))
    bm2b = jnp.broadcast_to(bm2[None, :], (8, 128))
    logit = _tc5(gsum, cnt, uva[:B], uvb[:B], uva[B:2 * B], uvb[B:2 * B],
                 Wm1, bm1b, wm2p, bm2b)
    return logit[:, 0]


def kernel(x, edge_index, batch_vec, u_idx, v_idx,
           W1, b1, W2, b2, W3, b3, Wm1, bm1, Wm2, bm2):
    return _impl(x, edge_index, batch_vec, u_idx, v_idx,
                 W1, b1, W2, b2, W3, b3, Wm1, bm1, Wm2, bm2)


# speedup vs baseline: 4.0805x; 1.0048x over previous
import functools

import jax
import jax.numpy as jnp
from jax import lax
from jax.experimental import pallas as pl
from jax.experimental.pallas import tpu as pltpu
from jax.experimental.pallas import tpu_sc as plsc

N = 10000
E = 320000
B = 512
DIN = 128
H = 256
HH = 128
NPAD = 10240
SENT = N
NC, NS = 2, 16
ECH = 128
NCH = 160
EPAD = NS * NCH * ECH
NHALF = NPAD // 2
ACC_R = NHALF + 8
PARK = NHALF
RPH = NHALF // NS
GPT = 128
RBLK = 512
NBLK = NPAD // RBLK

_f32 = jnp.float32
_i32 = jnp.int32


def _rewrite_dst(dst_v, j, idx_v, base, park):
    for k in range(ECH // 16):
        dvec = dst_v[j, pl.ds(k * 16, 16)]
        local = dvec - base
        ok = (local >= 0) & (local < NHALF)
        idx_v[pl.ds(k * 16, 16)] = jnp.where(ok, local, park)


def _rewrite_dst1(dst_v, idx_v, base, park):
    for k in range(ECH // 16):
        dvec = dst_v[pl.ds(k * 16, 16)]
        local = dvec - base
        ok = (local >= 0) & (local < NHALF)
        idx_v[pl.ds(k * 16, 16)] = jnp.where(ok, local, park)


def _deg_body(dst16, ones_hbm, deg, st0, st1, ones_v, idx_v, acc,
              dsem0, dsem1):
    c = lax.axis_index("c")
    s = lax.axis_index("s")
    base = c * NHALF
    row0 = base + s * RPH
    arow0 = s * RPH
    park = PARK + lax.rem(s, 8)
    stages = (st0, st1)
    dsems = (dsem0, dsem1)
    pltpu.sync_copy(ones_hbm.at[pl.ds(0, ECH)], ones_v)
    pltpu.sync_copy(ones_hbm.at[pl.ds(0, RPH)], acc.at[pl.ds(arow0, RPH)])
    plsc.subcore_barrier()
    pltpu.async_copy(dst16.at[s].at[0], st0, dsem0)

    def body(i, carry):
        for b in range(2):
            j = 2 * i + b

            @pl.when(j + 1 < NCH)
            def _pf():
                pltpu.async_copy(dst16.at[s].at[j + 1],
                                 stages[1 - b], dsems[1 - b])

            pltpu.make_async_copy(dst16.at[s].at[0],
                                  stages[b], dsems[b]).wait()
            _rewrite_dst1(stages[b], idx_v, base, park)
            pltpu.sync_copy(ones_v, acc.at[idx_v], add=True)
        return carry

    lax.fori_loop(0, NCH // 2, body, 0)
    plsc.subcore_barrier()
    pltpu.sync_copy(acc.at[pl.ds(arow0, RPH)], deg.at[pl.ds(row0, RPH)])


NBUF = 4


def _scatter_half(y, src_v, dstrow, stages, dsems, idx_v, gbufs, sems,
                  acc, base, park):
    for j in range(NBUF - 1):
        pltpu.async_copy(y.at[src_v.at[j]], gbufs[j], sems[j])
        pltpu.async_copy(dstrow.at[j], stages[j], dsems[j])

    def _chunk(j, b, prefetch):
        if prefetch:
            @pl.when(j + NBUF - 1 < NCH)
            def _pf():
                pltpu.async_copy(y.at[src_v.at[j + NBUF - 1]],
                                 gbufs[(b + NBUF - 1) % NBUF],
                                 sems[(b + NBUF - 1) % NBUF])
                pltpu.async_copy(dstrow.at[j + NBUF - 1],
                                 stages[(b + NBUF - 1) % NBUF],
                                 dsems[(b + NBUF - 1) % NBUF])

        pltpu.make_async_copy(y.at[pl.ds(0, ECH)],
                              gbufs[b], sems[b]).wait()
        pltpu.make_async_copy(dstrow.at[0],
                              stages[b], dsems[b]).wait()
        _rewrite_dst1(stages[b], idx_v, base, park)
        pltpu.sync_copy(gbufs[b], acc.at[idx_v], add=True)

    def body(i, carry):
        for b in range(NBUF):
            _chunk(NBUF * i + b, b, True)
        return carry

    nfull = NCH // NBUF
    lax.fori_loop(0, nfull, body, 0)
    for j in range(nfull * NBUF, NCH):
        _chunk(j, j % NBUF, False)


def _scatter_body(ya, yb, src16, dst16, outa, outb,
                  src_v, st0, st1, st2, st3, idx_v, gb0, gb1, gb2, gb3, acc,
                  sem0, sem1, sem2, sem3, dsem0, dsem1, dsem2, dsem3):
    c = lax.axis_index("c")
    s = lax.axis_index("s")
    arow0 = s * RPH
    park = PARK + lax.rem(s, 8)
    gbufs = (gb0, gb1, gb2, gb3)
    sems = (sem0, sem1, sem2, sem3)
    stages = (st0, st1, st2, st3)
    dsems = (dsem0, dsem1, dsem2, dsem3)
    dstrow = dst16.at[s]
    pltpu.sync_copy(src16.at[s], src_v)

    for p in range(2):
        base = p * NHALF
        row0 = base + s * RPH

        @pl.when(c == 0)
        def _i0(row0=row0):
            pltpu.sync_copy(ya.at[pl.ds(row0, RPH)], acc.at[pl.ds(arow0, RPH)])

        @pl.when(c == 1)
        def _i1(row0=row0):
            pltpu.sync_copy(yb.at[pl.ds(row0, RPH)], acc.at[pl.ds(arow0, RPH)])

        plsc.subcore_barrier()

        @pl.when(c == 0)
        def _l0(base=base):
            _scatter_half(ya, src_v, dstrow, stages, dsems, idx_v,
                          gbufs, sems, acc, base, park)

        @pl.when(c == 1)
        def _l1(base=base):
            _scatter_half(yb, src_v, dstrow, stages, dsems, idx_v,
                          gbufs, sems, acc, base, park)

        plsc.subcore_barrier()

        @pl.when(c == 0)
        def _o0(row0=row0):
            pltpu.sync_copy(acc.at[pl.ds(arow0, RPH)],
                            outa.at[pl.ds(row0, RPH)])

        @pl.when(c == 1)
        def _o1(row0=row0):
            pltpu.sync_copy(acc.at[pl.ds(arow0, RPH)],
                            outb.at[pl.ds(row0, RPH)])


def _gather_body(h3a, h3b, idx8, outa, outb, idx_v, rows_v, sem):
    c = lax.axis_index("c")
    s = lax.axis_index("s")

    @pl.when(s < 8)
    def _active():
        pltpu.sync_copy(idx8.at[s], idx_v)

        @pl.when(c == 0)
        def _g0():
            for t in range(2):
                row0 = s * GPT + t * (GPT // 2)
                pltpu.async_copy(
                    h3a.at[idx_v.at[pl.ds(t * (GPT // 2), GPT // 2)]],
                    rows_v, sem).wait()
                pltpu.sync_copy(rows_v, outa.at[pl.ds(row0, GPT // 2)])

        @pl.when(c == 1)
        def _g1():
            for t in range(2):
                row0 = s * GPT + t * (GPT // 2)
                pltpu.async_copy(
                    h3b.at[idx_v.at[pl.ds(t * (GPT // 2), GPT // 2)]],
                    rows_v, sem).wait()
                pltpu.sync_copy(rows_v, outb.at[pl.ds(row0, GPT // 2)])


@functools.lru_cache(maxsize=1)
def _sc_kernels():
    mesh = plsc.VectorSubcoreMesh(
        core_axis_name="c", subcore_axis_name="s",
        num_cores=NC, num_subcores=NS)
    deg = pl.kernel(
        _deg_body,
        out_type=jax.ShapeDtypeStruct((NPAD, HH), _f32),
        mesh=mesh,
        scratch_types=[
            pltpu.VMEM((ECH,), _i32),
            pltpu.VMEM((ECH,), _i32),
            pltpu.VMEM((ECH, HH), _f32),
            pltpu.VMEM((ECH,), _i32),
            pltpu.VMEM_SHARED((ACC_R, HH), _f32),
            pltpu.SemaphoreType.DMA,
            pltpu.SemaphoreType.DMA,
        ],
    )
    scat = pl.kernel(
        _scatter_body,
        out_type=(
            jax.ShapeDtypeStruct((NPAD, HH), _f32),
            jax.ShapeDtypeStruct((NPAD, HH), _f32),
        ),
        mesh=mesh,
        scratch_types=[
            pltpu.VMEM((NCH, ECH), _i32),
            pltpu.VMEM((ECH,), _i32),
            pltpu.VMEM((ECH,), _i32),
            pltpu.VMEM((ECH,), _i32),
            pltpu.VMEM((ECH,), _i32),
            pltpu.VMEM((ECH,), _i32),
            pltpu.VMEM((ECH, HH), _f32),
            pltpu.VMEM((ECH, HH), _f32),
            pltpu.VMEM((ECH, HH), _f32),
            pltpu.VMEM((ECH, HH), _f32),
            pltpu.VMEM_SHARED((ACC_R, HH), _f32),
            pltpu.SemaphoreType.DMA,
            pltpu.SemaphoreType.DMA,
            pltpu.SemaphoreType.DMA,
            pltpu.SemaphoreType.DMA,
            pltpu.SemaphoreType.DMA,
            pltpu.SemaphoreType.DMA,
            pltpu.SemaphoreType.DMA,
            pltpu.SemaphoreType.DMA,
        ],
    )
    gath = pl.kernel(
        _gather_body,
        out_type=(
            jax.ShapeDtypeStruct((8 * GPT, HH), _f32),
            jax.ShapeDtypeStruct((8 * GPT, HH), _f32),
        ),
        mesh=mesh,
        scratch_types=[
            pltpu.VMEM((GPT,), _i32),
            pltpu.VMEM((GPT // 2, HH), _f32),
            pltpu.SemaphoreType.DMA,
        ],
    )
    return deg, scat, gath


def _tc1_body(deg, x, w1, ya, yb, dis):
    d = lax.rsqrt(deg[:, :1])
    y = jnp.dot(x[...], w1[...], preferred_element_type=_f32) * d
    ya[...] = y[:, :HH]
    yb[...] = y[:, HH:]
    dis[...] = jnp.broadcast_to(d, (RBLK, 16))


_tc1 = pl.pallas_call(
    _tc1_body,
    grid=(NBLK,),
    in_specs=[
        pl.BlockSpec((RBLK, HH), lambda i: (i, 0)),
        pl.BlockSpec((RBLK, DIN), lambda i: (i, 0)),
        pl.BlockSpec((DIN, H), lambda i: (0, 0)),
    ],
    out_specs=[
        pl.BlockSpec((RBLK, HH), lambda i: (i, 0)),
        pl.BlockSpec((RBLK, HH), lambda i: (i, 0)),
        pl.BlockSpec((RBLK, 16), lambda i: (i, 0)),
    ],
    out_shape=[
        jax.ShapeDtypeStruct((NPAD, HH), _f32),
        jax.ShapeDtypeStruct((NPAD, HH), _f32),
        jax.ShapeDtypeStruct((NPAD, 16), _f32),
    ],
)


def _tcmid_body(sa, sb, dis, bias, w, ya, yb):
    d = dis[:, :1]
    ha = jnp.maximum(sa[...] * d + bias[:1, :HH], 0.0)
    hb = jnp.maximum(sb[...] * d + bias[:1, HH:], 0.0)
    h = jnp.concatenate([ha, hb], axis=1)
    y = jnp.dot(h, w[...], preferred_element_type=_f32) * d
    ya[...] = y[:, :HH]
    yb[...] = y[:, HH:]


_tcmid = pl.pallas_call(
    _tcmid_body,
    grid=(NBLK,),
    in_specs=[
        pl.BlockSpec((RBLK, HH), lambda i: (i, 0)),
        pl.BlockSpec((RBLK, HH), lambda i: (i, 0)),
        pl.BlockSpec((RBLK, 16), lambda i: (i, 0)),
        pl.BlockSpec((8, H), lambda i: (0, 0)),
        pl.BlockSpec((H, H), lambda i: (0, 0)),
    ],
    out_specs=[
        pl.BlockSpec((RBLK, HH), lambda i: (i, 0)),
        pl.BlockSpec((RBLK, HH), lambda i: (i, 0)),
    ],
    out_shape=[
        jax.ShapeDtypeStruct((NPAD, HH), _f32),
        jax.ShapeDtypeStruct((NPAD, HH), _f32),
    ],
)


def _tc4_body(sa, sb, dis, bias, batch, h3a_o, h3b_o, gsum_o, cnt_o):
    i = pl.program_id(0)
    d = dis[:, :1]
    ha = sa[...] * d + bias[:1, :HH]
    hb = sb[...] * d + bias[:1, HH:]
    h3a_o[...] = ha
    h3b_o[...] = hb
    bv = batch[:, :1]
    gid = lax.broadcasted_iota(_i32, (RBLK, B), 1)
    oh = (bv == gid).astype(_f32)
    h = jnp.concatenate([ha, hb], axis=1)
    pg = lax.dot_general(oh, h, (((0,), (0,)), ((), ())),
                         preferred_element_type=_f32)
    pc = lax.dot_general(
        oh, jnp.ones((RBLK, 16), _f32), (((0,), (0,)), ((), ())),
        preferred_element_type=_f32,
    )

    @pl.when(i == 0)
    def _z():
        gsum_o[...] = pg
        cnt_o[...] = pc

    @pl.when(i != 0)
    def _a():
        gsum_o[...] += pg
        cnt_o[...] += pc


_tc4 = pl.pallas_call(
    _tc4_body,
    grid=(NBLK,),
    in_specs=[
        pl.BlockSpec((RBLK, HH), lambda i: (i, 0)),
        pl.BlockSpec((RBLK, HH), lambda i: (i, 0)),
        pl.BlockSpec((RBLK, 16), lambda i: (i, 0)),
        pl.BlockSpec((8, H), lambda i: (0, 0)),
        pl.BlockSpec((RBLK, 16), lambda i: (i, 0)),
    ],
    out_specs=[
        pl.BlockSpec((RBLK, HH), lambda i: (i, 0)),
        pl.BlockSpec((RBLK, HH), lambda i: (i, 0)),
        pl.BlockSpec((B, H), lambda i: (0, 0)),
        pl.BlockSpec((B, 16), lambda i: (0, 0)),
    ],
    out_shape=[
        jax.ShapeDtypeStruct((NPAD, HH), _f32),
        jax.ShapeDtypeStruct((NPAD, HH), _f32),
        jax.ShapeDtypeStruct((B, H), _f32),
        jax.ShapeDtypeStruct((B, 16), _f32),
    ],
)


def _tc5_body(gsum, cnt, hua, hub, hva, hvb, wm1, bm1, wm2, bm2, out):
    g = gsum[...] / jnp.maximum(cnt[:, :1], 1.0)
    z = jnp.dot(g, wm1[0:H], preferred_element_type=_f32)
    z += jnp.dot(hua[...], wm1[H:H + HH], preferred_element_type=_f32)
    z += jnp.dot(hub[...], wm1[H + HH:2 * H], preferred_element_type=_f32)
    z += jnp.dot(hva[...], wm1[2 * H:2 * H + HH], preferred_element_type=_f32)
    z += jnp.dot(hvb[...], wm1[2 * H + HH:3 * H], preferred_element_type=_f32)
    z = jnp.maximum(z + bm1[:1, :], 0.0)
    out[...] = jnp.dot(z, wm2[...], preferred_element_type=_f32) + bm2[:1, :]


_tc5 = pl.pallas_call(
    _tc5_body,
    out_shape=jax.ShapeDtypeStruct((B, 128), _f32),
)


@jax.jit
def _impl(x, edge_index, batch_vec, u_idx, v_idx,
          W1, b1, W2, b2, W3, b3, Wm1, bm1, Wm2, bm2):
    src = edge_index[0].astype(_i32)
    dst = edge_index[1].astype(_i32)
    sent = jnp.full((EPAD - E,), SENT, _i32)
    src16 = jnp.concatenate([src, sent]).reshape(NS, NCH, ECH)
    dst16 = jnp.concatenate([dst, sent]).reshape(NS, NCH, ECH)
    xpad = jnp.pad(x.astype(_f32), ((0, NPAD - N), (0, 0)))
    ones_hbm = jnp.ones((NPAD, HH), _f32)

    deg_k, scat_k, gath_k = _sc_kernels()
    deg = deg_k(dst16, ones_hbm)
    y1a, y1b, dis = _tc1(deg, xpad, W1)

    b1b = jnp.broadcast_to(b1[None, :], (8, H))
    b2b = jnp.broadcast_to(b2[None, :], (8, H))
    b3b = jnp.broadcast_to(b3[None, :], (8, H))

    s1a, s1b = scat_k(y1a, y1b, src16, dst16)
    y2a, y2b = _tcmid(s1a, s1b, dis, b1b, W2)
    s2a, s2b = scat_k(y2a, y2b, src16, dst16)
    y3a, y3b = _tcmid(s2a, s2b, dis, b2b, W3)
    s3a, s3b = scat_k(y3a, y3b, src16, dst16)

    batch16 = jnp.broadcast_to(
        jnp.concatenate([batch_vec.astype(_i32),
                         jnp.full((NPAD - N,), B, _i32)])[:, None],
        (NPAD, 16))
    h3a, h3b, gsum, cnt = _tc4(s3a, s3b, dis, b3b, batch16)

    uv = jnp.concatenate(
        [u_idx.astype(_i32), v_idx.astype(_i32)]).reshape(8, GPT)
    uva, uvb = gath_k(h3a, h3b, uv)

    wm2p = jnp.pad(Wm2, ((0, 0), (0, 127)))
    bm1b = jnp.broadcast_to(bm1[None, :], (8, H))
    bm2b = jnp.broadcast_to(bm2[None, :], (8, 128))
    logit = _tc5(gsum, cnt, uva[:B], uvb[:B], uva[B:2 * B], uvb[B:2 * B],
                 Wm1, bm1b, wm2p, bm2b)
    return logit[:, 0]


def kernel(x, edge_index, batch_vec, u_idx, v_idx,
           W1, b1, W2, b2, W3, b3, Wm1, bm1, Wm2, bm2):
    return _impl(x, edge_index, batch_vec, u_idx, v_idx,
                 W1, b1, W2, b2, W3, b3, Wm1, bm1, Wm2, bm2)
